# Initial kernel scaffold; baseline (speedup 1.0000x reference)
import jax
import jax.numpy as jnp
from jax.experimental import pallas as pl


def kernel(query_emb, index_embs, Wq, bq, Wd, bd, Win, bin_, Wo, bo, W1, b1, W2, b2):
    raise NotImplementedError("write your pallas kernel here")


# speedup vs baseline: 7.6849x; 7.6849x over previous
import functools

import jax
import jax.numpy as jnp
from jax import lax
from jax.experimental import pallas as pl
from jax.experimental.pallas import tpu as pltpu
from jax.experimental.pallas import tpu_sc as plsc

_K = 128
_H = 8


def _topk_step(q_ref, e_ref, topv_ref, topi_ref, *, blk, k):
    j = pl.program_id(0)

    @pl.when(j == 0)
    def _init():
        topv_ref[...] = jnp.full(topv_ref.shape, -jnp.inf, jnp.float32)
        topi_ref[...] = jnp.zeros(topi_ref.shape, jnp.int32)

    s0 = lax.dot_general(q_ref[...], e_ref[...], (((1,), (1,)), ((), ())),
                         preferred_element_type=jnp.float32)
    b = s0.shape[0]
    iota_w = lax.broadcasted_iota(jnp.int32, (b, blk), 1)
    colk = lax.broadcasted_iota(jnp.int32, (b, k), 1)

    rv0 = topv_ref[...]
    ri0 = topi_ref[...]
    go0 = jnp.any(s0 > rv0[:, k - 1:k])

    def cond(c):
        return c[0]

    def body(c):
        _, rv, ri, s = c
        t = rv[:, k - 1:k]
        sm = jnp.where(s > t, s, -jnp.inf)
        v = jnp.max(sm, axis=1, keepdims=True)
        active = v > t
        pos = jnp.min(jnp.where(sm == v, iota_w, blk), axis=1, keepdims=True)
        gidx = j * blk + pos
        s = jnp.where((iota_w == pos) & active, -jnp.inf, s)
        p = jnp.sum((rv >= v).astype(jnp.int32), axis=1, keepdims=True)
        rv_sh = jnp.concatenate([rv[:, :1], rv[:, :k - 1]], axis=1)
        ri_sh = jnp.concatenate([ri[:, :1], ri[:, :k - 1]], axis=1)
        nrv = jnp.where(colk < p, rv, jnp.where(colk == p, v, rv_sh))
        nri = jnp.where(colk < p, ri, jnp.where(colk == p, gidx, ri_sh))
        rv = jnp.where(active, nrv, rv)
        ri = jnp.where(active, nri, ri)
        go = jnp.any(s > rv[:, k - 1:k])
        return go, rv, ri, s

    _, rv, ri, _ = lax.while_loop(cond, body, (go0, rv0, ri0, s0))
    topv_ref[...] = rv
    topi_ref[...] = ri


def _topk(query_emb, index_embs, blk=1000):
    n, d = index_embs.shape
    b = query_emb.shape[0]
    assert n % blk == 0
    return pl.pallas_call(
        functools.partial(_topk_step, blk=blk, k=_K),
        grid=(n // blk,),
        in_specs=[pl.BlockSpec((b, d), lambda j: (0, 0)),
                  pl.BlockSpec((blk, d), lambda j: (j, 0))],
        out_specs=[pl.BlockSpec((b, _K), lambda j: (0, 0)),
                   pl.BlockSpec((b, _K), lambda j: (0, 0))],
        out_shape=[jax.ShapeDtypeStruct((b, _K), jnp.float32),
                   jax.ShapeDtypeStruct((b, _K), jnp.int32)],
    )(query_emb, index_embs)


def _gather(index_embs, flat_idx):
    n, d = index_embs.shape
    bt = flat_idx.shape[0]
    nw = 32
    bpw = bt // nw
    mesh = plsc.VectorSubcoreMesh(core_axis_name="c", subcore_axis_name="s")

    @functools.partial(
        pl.kernel, mesh=mesh,
        out_type=jax.ShapeDtypeStruct((bt, d), jnp.float32),
        scratch_types=[pltpu.VMEM((bpw,), jnp.int32),
                       pltpu.VMEM((bpw, d), jnp.float32),
                       pltpu.SemaphoreType.DMA])
    def gk(table_hbm, idx_hbm, out_hbm, idx_v, rows_v, sem):
        wid = lax.axis_index("s") * 2 + lax.axis_index("c")
        base = wid * bpw
        pltpu.sync_copy(idx_hbm.at[pl.ds(base, bpw)], idx_v)
        pltpu.async_copy(table_hbm.at[idx_v], rows_v, sem).wait()
        pltpu.sync_copy(rows_v, out_hbm.at[pl.ds(base, bpw)])

    return gk(index_embs, flat_idx)


def _rerank_body(q_ref, cand_ref, wq_ref, bq_ref, wd_ref, bd_ref, win_ref,
                 bin_ref, wo_ref, bo_ref, w1_ref, b1_ref, w2_ref, b2_ref,
                 out_ref, *, b, k, d, h):
    dh = d // h

    def mm_t(x, w_row):
        return lax.dot_general(x, w_row, (((1,), (1,)), ((), ())),
                               preferred_element_type=jnp.float32)

    q = q_ref[...]
    cand = cand_ref[...]
    qx = mm_t(q, wq_ref[...]) + bq_ref[...][None, :]
    d_ = mm_t(cand, wd_ref[...]) + bd_ref[...][None, :]
    win = win_ref[...]
    binv = bin_ref[...]
    qp = mm_t(qx, win[0:d]) + binv[0:d][None, :]
    kp = mm_t(d_, win[d:2 * d]) + binv[d:2 * d][None, :]
    vp = mm_t(d_, win[2 * d:3 * d]) + binv[2 * d:3 * d][None, :]

    lane = lax.broadcasted_iota(jnp.int32, (d, h), 0)
    head = lax.broadcasted_iota(jnp.int32, (d, h), 1)
    a1 = ((lane // dh) == head).astype(jnp.float32)

    prod = (kp.reshape(b, k, d) * qp[:, None, :]).reshape(b * k, d)
    logits = lax.dot_general(prod, a1, (((1,), (0,)), ((), ())),
                             preferred_element_type=jnp.float32)
    lg = (logits * (1.0 / jnp.sqrt(jnp.float32(dh)))).reshape(b, k, h)
    mx = jnp.max(lg, axis=1, keepdims=True)
    ex = jnp.exp(lg - mx)
    att = (ex / jnp.sum(ex, axis=1, keepdims=True)).reshape(b * k, h)
    attb = lax.dot_general(att, a1, (((1,), (1,)), ((), ())),
                           preferred_element_type=jnp.float32)
    ov = jnp.sum((attb * vp).reshape(b, k, d), axis=1)
    o = mm_t(ov, wo_ref[...]) + bo_ref[...][None, :]
    h1 = jnp.maximum(mm_t(o, w1_ref[...]) + b1_ref[...][None, :], 0.0)
    rr = jnp.sum(h1 * w2_ref[...], axis=1, keepdims=True) + b2_ref[0]
    out_ref[...] = rr


def _rerank(query_emb, cand, Wq, bq, Wd, bd, Win, bin_, Wo, bo, W1, b1, W2, b2):
    b, d = query_emb.shape
    k = cand.shape[0] // b
    body = functools.partial(_rerank_body, b=b, k=k, d=d, h=_H)
    return pl.pallas_call(
        body,
        out_shape=jax.ShapeDtypeStruct((b, 1), jnp.float32),
    )(query_emb, cand, Wq, bq, Wd, bd, Win, bin_, Wo, bo, W1, b1, W2, b2)


def kernel(query_emb, index_embs, Wq, bq, Wd, bd, Win, bin_, Wo, bo, W1, b1,
           W2, b2):
    top_scores, top_idx = _topk(query_emb, index_embs)
    cand = _gather(index_embs, top_idx.reshape(-1))
    rr = _rerank(query_emb, cand, Wq, bq, Wd, bd, Win, bin_, Wo, bo, W1, b1,
                 W2, b2)
    return top_idx, top_scores, rr[:, 0]
